# baseline (speedup 1.0000x reference)
import functools

import jax
import jax.numpy as jnp
from jax import lax
from jax.experimental import pallas as pl
from jax.experimental.pallas import tpu as pltpu
from jax.experimental.pallas import tpu_sc as plsc

BATCH = 4096
SEQ = 200
D = 64
VOCAB = 1000000
NC, NS = 2, 16
NW = NC * NS
RPW = BATCH // NW
NBUF = 5
NROUNDS = SEQ // NBUF

_mesh = plsc.VectorSubcoreMesh(core_axis_name="c", subcore_axis_name="s")


@functools.partial(
    pl.kernel,
    mesh=_mesh,
    out_type=jax.ShapeDtypeStruct((BATCH, SEQ, 2 * D), jnp.float32),
    compiler_params=pltpu.CompilerParams(needs_layout_passes=False),
    scratch_types=[
        pltpu.VMEM((SEQ, RPW), jnp.int32),
        pltpu.VMEM((NBUF, RPW, 2 * D), jnp.float32),
        pltpu.SemaphoreType.DMA((NBUF,)),
        pltpu.SemaphoreType.DMA((NBUF,)),
    ],
)
def _emb_lookup(ids_hbm, table_hbm, out_hbm, ids_v, gbuf, gsem, ssem):
    wid = lax.axis_index("s") * NC + lax.axis_index("c")
    base = wid * RPW
    pltpu.sync_copy(ids_hbm.at[:, pl.ds(base, RPW)], ids_v)

    def gather(s, b):
        pltpu.async_copy(table_hbm.at[ids_v.at[s]], gbuf.at[b], gsem.at[b])

    def gather_wait(b):
        pltpu.make_async_copy(table_hbm.at[ids_v.at[0]], gbuf.at[b],
                              gsem.at[b]).wait()

    def store(s, b):
        pltpu.async_copy(gbuf.at[b], out_hbm.at[pl.ds(base, RPW), s],
                         ssem.at[b])

    def store_wait(b):
        pltpu.make_async_copy(gbuf.at[b], out_hbm.at[pl.ds(base, RPW), 0],
                              ssem.at[b]).wait()

    for b in range(NBUF):
        gather(b, b)

    def body(r, carry):
        s0 = r * NBUF
        for b in range(NBUF):
            gather_wait(b)
            store(s0 + b, b)
        for b in range(NBUF):
            store_wait(b)
            gather(s0 + NBUF + b, b)
        return carry

    lax.fori_loop(0, NROUNDS - 1, body, 0)

    s0 = (NROUNDS - 1) * NBUF
    for b in range(NBUF):
        gather_wait(b)
        store(s0 + b, b)
    for b in range(NBUF):
        store_wait(b)


def kernel(token_ids, embed_mat):
    padded = jnp.pad(embed_mat, ((0, 0), (0, D)))
    wide = _emb_lookup(token_ids.T, padded)
    return wide[:, :, :D]
